# baseline (speedup 1.0000x reference)
import functools

import jax
import jax.numpy as jnp
from jax import lax
from jax.experimental import pallas as pl
from jax.experimental.pallas import tpu as pltpu
from jax.experimental.pallas import tpu_sc as plsc

N = 10000
E = 320000
D = 128
DH = D // 2
DEN = DH
DPH = 80
NC, NS, L = 2, 16, 16
EPT = E // NS
K = 80
NCHUNK = EPT // K
NP = 10240
ROWS_PT = NP // NS


def _tc1_body(x_ref, emb_ref, w_ref, attij_ref, attemij_ref, haug_ref, a_ref):
    xb = x_ref[...]
    hb = jnp.dot(xb, w_ref[...], preferred_element_type=jnp.float32)
    haug_ref[...] = jnp.stack([hb[:, :DH], hb[:, DH:]], axis=0)
    a_ref[...] = (jnp.dot(hb, attij_ref[...], preferred_element_type=jnp.float32)
                  + jnp.dot(emb_ref[...], attemij_ref[...],
                            preferred_element_type=jnp.float32))


def _tc1(x, emb, w, att_ij, att_em_ij):
    bm = 1000
    grid = (N // bm,)
    return pl.pallas_call(
        _tc1_body,
        grid=grid,
        in_specs=[
            pl.BlockSpec((bm, D), lambda i: (i, 0)),
            pl.BlockSpec((bm, D), lambda i: (i, 0)),
            pl.BlockSpec((D, D), lambda i: (0, 0)),
            pl.BlockSpec((D, 2), lambda i: (0, 0)),
            pl.BlockSpec((D, 2), lambda i: (0, 0)),
        ],
        out_specs=[
            pl.BlockSpec((2, bm, DH), lambda i: (0, i, 0)),
            pl.BlockSpec((bm, 2), lambda i: (i, 0)),
        ],
        out_shape=[
            jax.ShapeDtypeStruct((2, N, DH), jnp.float32),
            jax.ShapeDtypeStruct((N, 2), jnp.float32),
        ],
    )(x, emb, w, att_ij, att_em_ij)


NB = 5


def _sc_edge_body(ei_hbm, cos_hbm, ai_hbm, aj_hbm, haug_hbm,
                  acc_hbm,
                  ai_v, aj_v, ei_v, cos_v, p_v, grow_v, srow_v, acc_sh,
                  isem, gsem, ssem):
    cid = lax.axis_index("c")
    sid = lax.axis_index("s")
    cid_off = lax.broadcast_in_dim(cid * N, (L,), ()).astype(jnp.int32)

    pltpu.sync_copy(ai_hbm, ai_v)
    pltpu.sync_copy(aj_hbm, aj_v)

    def _zrow(r, carry):
        for j in range(DPH // L):
            srow_v[0, r, pl.ds(j * L, L)] = jnp.zeros((L,), jnp.float32)
        return carry
    lax.fori_loop(0, K, _zrow, 0)

    zbase = sid * ROWS_PT
    def _zcp(cn, carry):
        pltpu.sync_copy(srow_v.at[0],
                        acc_sh.at[pl.ds(zbase + cn * K, K)])
        return carry
    lax.fori_loop(0, ROWS_PT // K, _zcp, 0)
    plsc.subcore_barrier()

    ebase = sid * EPT

    def idx_copies(c, b):
        cb = ebase + c * K
        return (pltpu.make_async_copy(ei_hbm.at[:, pl.ds(cb, K)], ei_v.at[b],
                                      isem.at[b]),
                pltpu.make_async_copy(cos_hbm.at[pl.ds(cb, K)], cos_v.at[b],
                                      isem.at[b]))

    def start_idx(c, b):
        for d in idx_copies(c, b):
            d.start()

    def wait_idx(c, b):
        for d in idx_copies(c, b):
            d.wait()

    def gather_copy(b):
        return pltpu.make_async_copy(haug_hbm.at[ei_v.at[b, 0]],
                                     grow_v.at[b], gsem.at[b])

    def scatter_copy(b):
        return pltpu.make_async_copy(srow_v.at[b],
                                     acc_sh.at[ei_v.at[b, 1]], ssem.at[b])

    def compute_p(b):
        for gi in range(K // L):
            sl = pl.ds(gi * L, L)
            s_idx = ei_v[b, 0, sl]
            d_idx = ei_v[b, 1, sl]
            al = (plsc.load_gather(ai_v, [d_idx])
                  + plsc.load_gather(aj_v, [s_idx])) * cos_v[b, sl]
            al = jnp.where(al >= 0.0, al, al * jnp.float32(0.2))
            p_v[b, sl] = jnp.exp(al)

    def shift_src(b):
        for gi in range(K // L):
            sl = pl.ds(gi * L, L)
            ei_v[b, 0, sl] = ei_v[b, 0, sl] + cid_off

    den_mask = (lax.broadcasted_iota(jnp.int32, (L,), 0)
                == 0).astype(jnp.float32)

    def scale_rows(b):
        def _scale(g, carry2):
            pvec = p_v[b, pl.ds(g * L, L)]
            for i in range(L):
                pb = lax.broadcast_in_dim(pvec[i], (L,), ())
                r = g * L + i
                for j in range(DH // L):
                    sj = pl.ds(j * L, L)
                    srow_v[b, r, sj] = grow_v[b, r, sj] * pb
                srow_v[b, r, pl.ds(DEN, L)] = den_mask * pb
            return carry2
        lax.fori_loop(0, K // L, _scale, 0)

    start_idx(0, 0)
    start_idx(1, 1)
    wait_idx(0, 0)
    compute_p(0)
    shift_src(0)
    gather_copy(0).start()

    def _round(t, carry):
        for r in range(NB):
            c = t * NB + r
            r1 = (r + 1) % NB
            r2 = (r + 2) % NB

            @pl.when(c >= NB - 2)
            def _():
                scatter_copy(r2).wait()

            @pl.when(c + 2 < NCHUNK)
            def _():
                start_idx(c + 2, r2)

            @pl.when(c + 1 < NCHUNK)
            def _():
                wait_idx(c + 1, r1)
                compute_p(r1)
                shift_src(r1)
                gather_copy(r1).start()

            gather_copy(r).wait()
            scale_rows(r)
            pltpu.async_copy(srow_v.at[r], acc_sh.at[ei_v.at[r, 1]],
                             ssem.at[r], add=True)
        return carry
    lax.fori_loop(0, NCHUNK // NB, _round, 0)

    for b in ((NCHUNK - 3) % NB, (NCHUNK - 2) % NB, (NCHUNK - 1) % NB):
        scatter_copy(b).wait()

    plsc.subcore_barrier()
    pltpu.sync_copy(acc_sh.at[pl.ds(sid * ROWS_PT, ROWS_PT)],
                    acc_hbm.at[cid, pl.ds(sid * ROWS_PT, ROWS_PT)])


@functools.cache
def _make_sc_edge():
    return functools.partial(
        pl.kernel,
        out_type=jax.ShapeDtypeStruct((NC, NP, DPH), jnp.float32),
        mesh=plsc.VectorSubcoreMesh(core_axis_name="c", subcore_axis_name="s",
                                    num_cores=NC, num_subcores=NS),
        compiler_params=pltpu.CompilerParams(needs_layout_passes=False,
                                             use_tc_tiling_on_sc=False),
        scratch_types=[
            pltpu.VMEM((N,), jnp.float32),
            pltpu.VMEM((N,), jnp.float32),
            pltpu.VMEM((NB, 2, K), jnp.int32),
            pltpu.VMEM((NB, K), jnp.float32),
            pltpu.VMEM((NB, K), jnp.float32),
            pltpu.VMEM((NB, K, DH), jnp.float32),
            pltpu.VMEM((NB, K, DPH), jnp.float32),
            pltpu.VMEM_SHARED((NP, DPH), jnp.float32),
            pltpu.SemaphoreType.DMA((NB,)),
            pltpu.SemaphoreType.DMA((NB,)),
            pltpu.SemaphoreType.DMA((NB,)),
        ],
    )(_sc_edge_body)


def _tc2a_body(acc_ref, bias_ref, out0_ref, sum_ref, sq_ref):
    i = pl.program_id(0)
    a0 = acc_ref[0]
    a1 = acc_ref[1]
    num = jnp.concatenate([a0[:, :DH], a1[:, :DH]], axis=1)
    den = a0[:, DEN:DEN + 1]
    o = num / (den + 1e-16) + bias_ref[...]
    out0_ref[...] = o

    @pl.when(i == 0)
    def _():
        sum_ref[...] = jnp.zeros_like(sum_ref)
        sq_ref[...] = jnp.zeros_like(sq_ref)

    sum_ref[...] += jnp.sum(o, axis=0, keepdims=True)
    sq_ref[...] += jnp.sum(o * o, axis=0, keepdims=True)


def _tc2a(acc, bias_row):
    bm = 1000
    return pl.pallas_call(
        _tc2a_body,
        grid=(N // bm,),
        in_specs=[
            pl.BlockSpec((NC, bm, DPH), lambda i: (0, i, 0)),
            pl.BlockSpec((1, D), lambda i: (0, 0)),
        ],
        out_specs=[
            pl.BlockSpec((bm, D), lambda i: (i, 0)),
            pl.BlockSpec((1, D), lambda i: (0, 0)),
            pl.BlockSpec((1, D), lambda i: (0, 0)),
        ],
        out_shape=[
            jax.ShapeDtypeStruct((N, D), jnp.float32),
            jax.ShapeDtypeStruct((1, D), jnp.float32),
            jax.ShapeDtypeStruct((1, D), jnp.float32),
        ],
    )(acc, bias_row)


def _tc2b_body(out0_ref, sum_ref, sq_ref, gamma_ref, beta_ref, y_ref):
    mean = sum_ref[...] / jnp.float32(N)
    var = sq_ref[...] / jnp.float32(N) - mean * mean
    inv = lax.rsqrt(var + 1e-5)
    y = (out0_ref[...] - mean) * inv * gamma_ref[...] + beta_ref[...]
    y_ref[...] = jnp.maximum(y, 0.0)


def _tc2b(out0, s, q, gamma_row, beta_row):
    bm = 1000
    return pl.pallas_call(
        _tc2b_body,
        grid=(N // bm,),
        in_specs=[
            pl.BlockSpec((bm, D), lambda i: (i, 0)),
            pl.BlockSpec((1, D), lambda i: (0, 0)),
            pl.BlockSpec((1, D), lambda i: (0, 0)),
            pl.BlockSpec((1, D), lambda i: (0, 0)),
            pl.BlockSpec((1, D), lambda i: (0, 0)),
        ],
        out_specs=pl.BlockSpec((bm, D), lambda i: (i, 0)),
        out_shape=jax.ShapeDtypeStruct((N, D), jnp.float32),
    )(out0, s, q, gamma_row, beta_row)


def kernel(x, edge_index, cos_topk, embedding, W, att_i, att_j,
           att_em_i, att_em_j, bias, gamma, beta):
    att_ij = jnp.stack([att_i, att_j], axis=1)
    att_em_ij = jnp.stack([att_em_i, att_em_j], axis=1)
    haug, a2 = _tc1(x, embedding, W, att_ij, att_em_ij)
    ai = a2[:, 0]
    aj = a2[:, 1]
    acc = _make_sc_edge()(edge_index, cos_topk, ai, aj,
                          haug.reshape(2 * N, DH))
    out0, s, q = _tc2a(acc, bias.reshape(1, D))
    return _tc2b(out0, s, q, gamma.reshape(1, D), beta.reshape(1, D))


# speedup vs baseline: 1.8658x; 1.8658x over previous
import functools

import jax
import jax.numpy as jnp
from jax import lax
from jax.experimental import pallas as pl
from jax.experimental.pallas import tpu as pltpu
from jax.experimental.pallas import tpu_sc as plsc

N = 10000
E = 320000
D = 128
DH = D // 2
DEN = DH
DPH = 80
NC, NS, L = 2, 16, 16
EPT = E // NS
K = 80
NCHUNK = EPT // K
NP = 10240
ROWS_PT = NP // NS


def _tc1_body(x_ref, emb_ref, w_ref, attij_ref, attemij_ref, haug_ref, a_ref):
    xb = x_ref[...]
    hb = jnp.dot(xb, w_ref[...], preferred_element_type=jnp.float32)
    ones_col = (lax.broadcasted_iota(jnp.int32, (xb.shape[0], DPH - DH), 1)
                == 0).astype(jnp.float32)
    haug_ref[...] = jnp.stack(
        [jnp.concatenate([hb[:, :DH], ones_col], axis=1),
         jnp.concatenate([hb[:, DH:], ones_col], axis=1)], axis=0)
    a_ref[...] = (jnp.dot(hb, attij_ref[...], preferred_element_type=jnp.float32)
                  + jnp.dot(emb_ref[...], attemij_ref[...],
                            preferred_element_type=jnp.float32))


def _tc1(x, emb, w, att_ij, att_em_ij):
    bm = 1000
    grid = (N // bm,)
    return pl.pallas_call(
        _tc1_body,
        grid=grid,
        in_specs=[
            pl.BlockSpec((bm, D), lambda i: (i, 0)),
            pl.BlockSpec((bm, D), lambda i: (i, 0)),
            pl.BlockSpec((D, D), lambda i: (0, 0)),
            pl.BlockSpec((D, 2), lambda i: (0, 0)),
            pl.BlockSpec((D, 2), lambda i: (0, 0)),
        ],
        out_specs=[
            pl.BlockSpec((2, bm, DPH), lambda i: (0, i, 0)),
            pl.BlockSpec((bm, 2), lambda i: (i, 0)),
        ],
        out_shape=[
            jax.ShapeDtypeStruct((2, N, DPH), jnp.float32),
            jax.ShapeDtypeStruct((N, 2), jnp.float32),
        ],
    )(x, emb, w, att_ij, att_em_ij)


NB = 5


def _sc_edge_body(ei_hbm, cos_hbm, ai_hbm, aj_hbm, haug_hbm,
                  acc_hbm,
                  ai_v, aj_v, ei_v, cos_v, p_v, rows_v, acc_sh,
                  isem, gsem, ssem):
    cid = lax.axis_index("c")
    sid = lax.axis_index("s")
    cid_off = lax.broadcast_in_dim(cid * N, (L,), ()).astype(jnp.int32)

    pltpu.sync_copy(ai_hbm, ai_v)
    pltpu.sync_copy(aj_hbm, aj_v)

    def _zrow(r, carry):
        for j in range(DPH // L):
            rows_v[0, r, pl.ds(j * L, L)] = jnp.zeros((L,), jnp.float32)
        return carry
    lax.fori_loop(0, K, _zrow, 0)

    zbase = sid * ROWS_PT
    def _zcp(cn, carry):
        pltpu.sync_copy(rows_v.at[0],
                        acc_sh.at[pl.ds(zbase + cn * K, K)])
        return carry
    lax.fori_loop(0, ROWS_PT // K, _zcp, 0)
    plsc.subcore_barrier()

    ebase = sid * EPT

    def idx_copies(c, b):
        cb = ebase + c * K
        return (pltpu.make_async_copy(ei_hbm.at[:, pl.ds(cb, K)], ei_v.at[b],
                                      isem.at[b]),
                pltpu.make_async_copy(cos_hbm.at[pl.ds(cb, K)], cos_v.at[b],
                                      isem.at[b]))

    def start_idx(c, b):
        for d in idx_copies(c, b):
            d.start()

    def wait_idx(c, b):
        for d in idx_copies(c, b):
            d.wait()

    def gather_copy(b):
        return pltpu.make_async_copy(haug_hbm.at[ei_v.at[b, 0]],
                                     rows_v.at[b], gsem.at[b])

    def scatter_copy(b):
        return pltpu.make_async_copy(rows_v.at[b],
                                     acc_sh.at[ei_v.at[b, 1]], ssem.at[b])

    def compute_p(b):
        for gi in range(K // L):
            sl = pl.ds(gi * L, L)
            s_idx = ei_v[b, 0, sl]
            d_idx = ei_v[b, 1, sl]
            al = (plsc.load_gather(ai_v, [d_idx])
                  + plsc.load_gather(aj_v, [s_idx])) * cos_v[b, sl]
            al = jnp.where(al >= 0.0, al, al * jnp.float32(0.2))
            p_v[b, sl] = jnp.exp(al)

    def shift_src(b):
        for gi in range(K // L):
            sl = pl.ds(gi * L, L)
            ei_v[b, 0, sl] = ei_v[b, 0, sl] + cid_off

    def scale_rows(b):
        def _scale(g, carry2):
            pvec = p_v[b, pl.ds(g * L, L)]
            for i in range(L):
                pb = lax.broadcast_in_dim(pvec[i], (L,), ())
                r = g * L + i
                for j in range(DPH // L):
                    sj = pl.ds(j * L, L)
                    rows_v[b, r, sj] = rows_v[b, r, sj] * pb
            return carry2
        lax.fori_loop(0, K // L, _scale, 0)

    start_idx(0, 0)
    start_idx(1, 1)
    wait_idx(0, 0)
    compute_p(0)
    shift_src(0)
    gather_copy(0).start()
    wait_idx(1, 1)
    compute_p(1)
    shift_src(1)
    gather_copy(1).start()

    def _round(t, carry):
        for r in range(NB):
            c = t * NB + r
            r2 = (r + 2) % NB

            @pl.when(c >= NB - 2)
            def _():
                scatter_copy(r2).wait()

            @pl.when(c + 2 < NCHUNK)
            def _():
                start_idx(c + 2, r2)

            gather_copy(r).wait()
            scale_rows(r)
            pltpu.async_copy(rows_v.at[r], acc_sh.at[ei_v.at[r, 1]],
                             ssem.at[r], add=True)

            @pl.when(c + 2 < NCHUNK)
            def _():
                wait_idx(c + 2, r2)
                compute_p(r2)
                shift_src(r2)
                gather_copy(r2).start()
        return carry
    lax.fori_loop(0, NCHUNK // NB, _round, 0)

    for b in ((NCHUNK - 3) % NB, (NCHUNK - 2) % NB, (NCHUNK - 1) % NB):
        scatter_copy(b).wait()

    plsc.subcore_barrier()
    pltpu.sync_copy(acc_sh.at[pl.ds(sid * ROWS_PT, ROWS_PT)],
                    acc_hbm.at[cid, pl.ds(sid * ROWS_PT, ROWS_PT)])


@functools.cache
def _make_sc_edge():
    return functools.partial(
        pl.kernel,
        out_type=jax.ShapeDtypeStruct((NC, NP, DPH), jnp.float32),
        mesh=plsc.VectorSubcoreMesh(core_axis_name="c", subcore_axis_name="s",
                                    num_cores=NC, num_subcores=NS),
        compiler_params=pltpu.CompilerParams(needs_layout_passes=False,
                                             use_tc_tiling_on_sc=False),
        scratch_types=[
            pltpu.VMEM((N,), jnp.float32),
            pltpu.VMEM((N,), jnp.float32),
            pltpu.VMEM((NB, 2, K), jnp.int32),
            pltpu.VMEM((NB, K), jnp.float32),
            pltpu.VMEM((NB, K), jnp.float32),
            pltpu.VMEM((NB, K, DPH), jnp.float32),
            pltpu.VMEM_SHARED((NP, DPH), jnp.float32),
            pltpu.SemaphoreType.DMA((NB,)),
            pltpu.SemaphoreType.DMA((NB,)),
            pltpu.SemaphoreType.DMA((NB,)),
        ],
    )(_sc_edge_body)


def _tc2a_body(acc_ref, bias_ref, out0_ref, sum_ref, sq_ref):
    i = pl.program_id(0)
    a0 = acc_ref[0]
    a1 = acc_ref[1]
    num = jnp.concatenate([a0[:, :DH], a1[:, :DH]], axis=1)
    den = a0[:, DEN:DEN + 1]
    o = num / (den + 1e-16) + bias_ref[...]
    out0_ref[...] = o

    @pl.when(i == 0)
    def _():
        sum_ref[...] = jnp.zeros_like(sum_ref)
        sq_ref[...] = jnp.zeros_like(sq_ref)

    sum_ref[...] += jnp.sum(o, axis=0, keepdims=True)
    sq_ref[...] += jnp.sum(o * o, axis=0, keepdims=True)


def _tc2a(acc, bias_row):
    bm = 1000
    return pl.pallas_call(
        _tc2a_body,
        grid=(N // bm,),
        in_specs=[
            pl.BlockSpec((NC, bm, DPH), lambda i: (0, i, 0)),
            pl.BlockSpec((1, D), lambda i: (0, 0)),
        ],
        out_specs=[
            pl.BlockSpec((bm, D), lambda i: (i, 0)),
            pl.BlockSpec((1, D), lambda i: (0, 0)),
            pl.BlockSpec((1, D), lambda i: (0, 0)),
        ],
        out_shape=[
            jax.ShapeDtypeStruct((N, D), jnp.float32),
            jax.ShapeDtypeStruct((1, D), jnp.float32),
            jax.ShapeDtypeStruct((1, D), jnp.float32),
        ],
    )(acc, bias_row)


def _tc2b_body(out0_ref, sum_ref, sq_ref, gamma_ref, beta_ref, y_ref):
    mean = sum_ref[...] / jnp.float32(N)
    var = sq_ref[...] / jnp.float32(N) - mean * mean
    inv = lax.rsqrt(var + 1e-5)
    y = (out0_ref[...] - mean) * inv * gamma_ref[...] + beta_ref[...]
    y_ref[...] = jnp.maximum(y, 0.0)


def _tc2b(out0, s, q, gamma_row, beta_row):
    bm = 1000
    return pl.pallas_call(
        _tc2b_body,
        grid=(N // bm,),
        in_specs=[
            pl.BlockSpec((bm, D), lambda i: (i, 0)),
            pl.BlockSpec((1, D), lambda i: (0, 0)),
            pl.BlockSpec((1, D), lambda i: (0, 0)),
            pl.BlockSpec((1, D), lambda i: (0, 0)),
            pl.BlockSpec((1, D), lambda i: (0, 0)),
        ],
        out_specs=pl.BlockSpec((bm, D), lambda i: (i, 0)),
        out_shape=jax.ShapeDtypeStruct((N, D), jnp.float32),
    )(out0, s, q, gamma_row, beta_row)


def kernel(x, edge_index, cos_topk, embedding, W, att_i, att_j,
           att_em_i, att_em_j, bias, gamma, beta):
    att_ij = jnp.stack([att_i, att_j], axis=1)
    att_em_ij = jnp.stack([att_em_i, att_em_j], axis=1)
    haug, a2 = _tc1(x, embedding, W, att_ij, att_em_ij)
    ai = a2[:, 0]
    aj = a2[:, 1]
    acc = _make_sc_edge()(edge_index, cos_topk, ai, aj,
                          haug.reshape(2 * N, DPH))
    out0, s, q = _tc2a(acc, bias.reshape(1, D))
    return _tc2b(out0, s, q, gamma.reshape(1, D), beta.reshape(1, D))


# speedup vs baseline: 1.8985x; 1.0175x over previous
import functools

import jax
import jax.numpy as jnp
from jax import lax
from jax.experimental import pallas as pl
from jax.experimental.pallas import tpu as pltpu
from jax.experimental.pallas import tpu_sc as plsc

N = 10000
E = 320000
D = 128
DH = D // 2
DEN = DH
DPH = 80
NC, NS, L = 2, 16, 16
EPT = E // NS
K = 80
NCHUNK = EPT // K
NP = 10240
ROWS_PT = NP // NS


def _tc1_body(x_ref, emb_ref, w_ref, attij_ref, attemij_ref, haug_ref, a_ref):
    xb = x_ref[...]
    hb = jnp.dot(xb, w_ref[...], preferred_element_type=jnp.float32)
    ones_col = (lax.broadcasted_iota(jnp.int32, (xb.shape[0], DPH - DH), 1)
                == 0).astype(jnp.float32)
    haug_ref[...] = jnp.stack(
        [jnp.concatenate([hb[:, :DH], ones_col], axis=1),
         jnp.concatenate([hb[:, DH:], ones_col], axis=1)], axis=0)
    a_ref[...] = (jnp.dot(hb, attij_ref[...], preferred_element_type=jnp.float32)
                  + jnp.dot(emb_ref[...], attemij_ref[...],
                            preferred_element_type=jnp.float32))


def _tc1(x, emb, w, att_ij, att_em_ij):
    bm = 1000
    grid = (N // bm,)
    return pl.pallas_call(
        _tc1_body,
        grid=grid,
        in_specs=[
            pl.BlockSpec((bm, D), lambda i: (i, 0)),
            pl.BlockSpec((bm, D), lambda i: (i, 0)),
            pl.BlockSpec((D, D), lambda i: (0, 0)),
            pl.BlockSpec((D, 2), lambda i: (0, 0)),
            pl.BlockSpec((D, 2), lambda i: (0, 0)),
        ],
        out_specs=[
            pl.BlockSpec((2, bm, DPH), lambda i: (0, i, 0)),
            pl.BlockSpec((bm, 2), lambda i: (i, 0)),
        ],
        out_shape=[
            jax.ShapeDtypeStruct((2, N, DPH), jnp.float32),
            jax.ShapeDtypeStruct((N, 2), jnp.float32),
        ],
    )(x, emb, w, att_ij, att_em_ij)


NB = 5


def _sc_edge_body(ei_hbm, cos_hbm, ai_hbm, aj_hbm, haug_hbm,
                  acc_hbm,
                  ai_v, aj_v, ei_v, cos_v, p_v, rows_v, acc_sh,
                  isem, gsem, ssem):
    cid = lax.axis_index("c")
    sid = lax.axis_index("s")
    cid_off = lax.broadcast_in_dim(cid * N, (L,), ()).astype(jnp.int32)

    pltpu.sync_copy(ai_hbm, ai_v)
    pltpu.sync_copy(aj_hbm, aj_v)

    def _zrow(r, carry):
        for j in range(DPH // L):
            rows_v[0, r, pl.ds(j * L, L)] = jnp.zeros((L,), jnp.float32)
        return carry
    lax.fori_loop(0, K, _zrow, 0)

    zbase = sid * ROWS_PT
    def _zcp(cn, carry):
        pltpu.sync_copy(rows_v.at[0],
                        acc_sh.at[pl.ds(zbase + cn * K, K)])
        return carry
    lax.fori_loop(0, ROWS_PT // K, _zcp, 0)
    plsc.subcore_barrier()

    ebase = sid * EPT

    def idx_copies(c, b):
        cb = ebase + c * K
        return (pltpu.make_async_copy(ei_hbm.at[:, pl.ds(cb, K)], ei_v.at[b],
                                      isem.at[b]),
                pltpu.make_async_copy(cos_hbm.at[pl.ds(cb, K)], cos_v.at[b],
                                      isem.at[b]))

    def start_idx(c, b):
        for d in idx_copies(c, b):
            d.start()

    def wait_idx(c, b):
        for d in idx_copies(c, b):
            d.wait()

    def gather_copy(b):
        return pltpu.make_async_copy(haug_hbm.at[ei_v.at[b, 0]],
                                     rows_v.at[b], gsem.at[b])

    def scatter_copy(b):
        return pltpu.make_async_copy(rows_v.at[b],
                                     acc_sh.at[ei_v.at[b, 1]], ssem.at[b])

    def compute_p(b):
        for gi in range(K // L):
            sl = pl.ds(gi * L, L)
            s_idx = ei_v[b, 0, sl]
            d_idx = ei_v[b, 1, sl]
            al = (plsc.load_gather(ai_v, [d_idx])
                  + plsc.load_gather(aj_v, [s_idx])) * cos_v[b, sl]
            al = jnp.where(al >= 0.0, al, al * jnp.float32(0.2))
            p_v[b, sl] = jnp.exp(al)

    def shift_src(b):
        for gi in range(K // L):
            sl = pl.ds(gi * L, L)
            ei_v[b, 0, sl] = ei_v[b, 0, sl] + cid_off

    def scale_rows(b):
        def _scale(g, carry2):
            pvec = p_v[b, pl.ds(g * L, L)]
            for i in range(L):
                pb = lax.broadcast_in_dim(pvec[i], (L,), ())
                r = g * L + i
                for j in range(DPH // L):
                    sj = pl.ds(j * L, L)
                    rows_v[b, r, sj] = rows_v[b, r, sj] * pb
            return carry2
        lax.fori_loop(0, K // L, _scale, 0)

    start_idx(0, 0)
    start_idx(1, 1)
    wait_idx(0, 0)
    compute_p(0)
    shift_src(0)
    gather_copy(0).start()
    wait_idx(1, 1)
    compute_p(1)
    shift_src(1)
    gather_copy(1).start()

    def _round(t, carry):
        for r in range(NB):
            c = t * NB + r
            r2 = (r + 2) % NB

            @pl.when(c >= NB - 2)
            def _():
                scatter_copy(r2).wait()

            @pl.when(c + 2 < NCHUNK)
            def _():
                start_idx(c + 2, r2)

            gather_copy(r).wait()
            scale_rows(r)
            pltpu.async_copy(rows_v.at[r], acc_sh.at[ei_v.at[r, 1]],
                             ssem.at[r], add=True)

            @pl.when(c + 2 < NCHUNK)
            def _():
                wait_idx(c + 2, r2)
                compute_p(r2)
                shift_src(r2)
                gather_copy(r2).start()
        return carry
    lax.fori_loop(0, NCHUNK // NB, _round, 0)

    for b in ((NCHUNK - 3) % NB, (NCHUNK - 2) % NB, (NCHUNK - 1) % NB):
        scatter_copy(b).wait()

    plsc.subcore_barrier()
    pltpu.sync_copy(acc_sh.at[pl.ds(sid * ROWS_PT, ROWS_PT)],
                    acc_hbm.at[cid, pl.ds(sid * ROWS_PT, ROWS_PT)])


@functools.cache
def _make_sc_edge():
    return functools.partial(
        pl.kernel,
        out_type=jax.ShapeDtypeStruct((NC, NP, DPH), jnp.float32),
        mesh=plsc.VectorSubcoreMesh(core_axis_name="c", subcore_axis_name="s",
                                    num_cores=NC, num_subcores=NS),
        compiler_params=pltpu.CompilerParams(needs_layout_passes=False,
                                             use_tc_tiling_on_sc=False),
        scratch_types=[
            pltpu.VMEM((N,), jnp.float32),
            pltpu.VMEM((N,), jnp.float32),
            pltpu.VMEM((NB, 2, K), jnp.int32),
            pltpu.VMEM((NB, K), jnp.float32),
            pltpu.VMEM((NB, K), jnp.float32),
            pltpu.VMEM((NB, K, DPH), jnp.float32),
            pltpu.VMEM_SHARED((NP, DPH), jnp.float32),
            pltpu.SemaphoreType.DMA((NB,)),
            pltpu.SemaphoreType.DMA((NB,)),
            pltpu.SemaphoreType.DMA((NB,)),
        ],
    )(_sc_edge_body)


def _tc2_body(acc_ref, bias_ref, gamma_ref, beta_ref, y_ref,
              out0_ref, sum_ref, sq_ref):
    p = pl.program_id(0)
    i = pl.program_id(1)

    @pl.when(p == 0)
    def _():
        a0 = acc_ref[0]
        a1 = acc_ref[1]
        num = jnp.concatenate([a0[:, :DH], a1[:, :DH]], axis=1)
        den = a0[:, DEN:DEN + 1]
        o = num / (den + 1e-16) + bias_ref[...]
        out0_ref[pl.ds(i * y_ref.shape[0], y_ref.shape[0]), :] = o

        @pl.when(i == 0)
        def _():
            sum_ref[...] = jnp.zeros_like(sum_ref)
            sq_ref[...] = jnp.zeros_like(sq_ref)

        sum_ref[...] += jnp.sum(o, axis=0, keepdims=True)
        sq_ref[...] += jnp.sum(o * o, axis=0, keepdims=True)

    @pl.when(p == 1)
    def _():
        mean = sum_ref[...] / jnp.float32(N)
        var = sq_ref[...] / jnp.float32(N) - mean * mean
        inv = lax.rsqrt(var + 1e-5)
        o = out0_ref[pl.ds(i * y_ref.shape[0], y_ref.shape[0]), :]
        y = (o - mean) * inv * gamma_ref[...] + beta_ref[...]
        y_ref[...] = jnp.maximum(y, 0.0)


def _tc2(acc, bias_row, gamma_row, beta_row):
    bm = 1000
    return pl.pallas_call(
        _tc2_body,
        grid=(2, N // bm),
        in_specs=[
            pl.BlockSpec((NC, bm, DPH), lambda p, i: (0, i * (1 - p), 0)),
            pl.BlockSpec((1, D), lambda p, i: (0, 0)),
            pl.BlockSpec((1, D), lambda p, i: (0, 0)),
            pl.BlockSpec((1, D), lambda p, i: (0, 0)),
        ],
        out_specs=pl.BlockSpec((bm, D), lambda p, i: (i * p, 0)),
        out_shape=jax.ShapeDtypeStruct((N, D), jnp.float32),
        scratch_shapes=[
            pltpu.VMEM((N, D), jnp.float32),
            pltpu.VMEM((1, D), jnp.float32),
            pltpu.VMEM((1, D), jnp.float32),
        ],
    )(acc, bias_row, gamma_row, beta_row)


def kernel(x, edge_index, cos_topk, embedding, W, att_i, att_j,
           att_em_i, att_em_j, bias, gamma, beta):
    att_ij = jnp.stack([att_i, att_j], axis=1)
    att_em_ij = jnp.stack([att_em_i, att_em_j], axis=1)
    haug, a2 = _tc1(x, embedding, W, att_ij, att_em_ij)
    ai = a2[:, 0]
    aj = a2[:, 1]
    acc = _make_sc_edge()(edge_index, cos_topk, ai, aj,
                          haug.reshape(2 * N, DPH))
    return _tc2(acc, bias.reshape(1, D), gamma.reshape(1, D),
                beta.reshape(1, D))


# speedup vs baseline: 1.9236x; 1.0132x over previous
import functools

import jax
import jax.numpy as jnp
from jax import lax
from jax.experimental import pallas as pl
from jax.experimental.pallas import tpu as pltpu
from jax.experimental.pallas import tpu_sc as plsc

N = 10000
E = 320000
D = 128
DH = D // 2
DEN = DH
DPH = 80
NC, NS, L = 2, 16, 16
EPT = E // NS
K = 80
NCHUNK = EPT // K
NP = 10240
ROWS_PT = NP // NS


def _tc1_body(x_ref, emb_ref, w_ref, attij_ref, attemij_ref, haug_ref, a_ref):
    xb = x_ref[...]
    hb = jnp.dot(xb, w_ref[...], preferred_element_type=jnp.float32)
    ones_col = (lax.broadcasted_iota(jnp.int32, (xb.shape[0], DPH - DH), 1)
                == 0).astype(jnp.float32)
    haug_ref[...] = jnp.stack(
        [jnp.concatenate([hb[:, :DH], ones_col], axis=1),
         jnp.concatenate([hb[:, DH:], ones_col], axis=1)], axis=0)
    a_ref[...] = (jnp.dot(hb, attij_ref[...], preferred_element_type=jnp.float32)
                  + jnp.dot(emb_ref[...], attemij_ref[...],
                            preferred_element_type=jnp.float32))


def _tc1(x, emb, w, att_ij, att_em_ij):
    bm = 1000
    grid = (N // bm,)
    return pl.pallas_call(
        _tc1_body,
        grid=grid,
        in_specs=[
            pl.BlockSpec((bm, D), lambda i: (i, 0)),
            pl.BlockSpec((bm, D), lambda i: (i, 0)),
            pl.BlockSpec((D, D), lambda i: (0, 0)),
            pl.BlockSpec((D, 2), lambda i: (0, 0)),
            pl.BlockSpec((D, 2), lambda i: (0, 0)),
        ],
        out_specs=[
            pl.BlockSpec((2, bm, DPH), lambda i: (0, i, 0)),
            pl.BlockSpec((bm, 2), lambda i: (i, 0)),
        ],
        out_shape=[
            jax.ShapeDtypeStruct((2, N, DPH), jnp.float32),
            jax.ShapeDtypeStruct((N, 2), jnp.float32),
        ],
    )(x, emb, w, att_ij, att_em_ij)


NB = 5


def _sc_edge_body(ei_hbm, cos_hbm, ai_hbm, aj_hbm, haug_hbm,
                  acc_hbm,
                  ai_v, aj_v, ei_v, cos_v, p_v, rows_v, acc_sh,
                  isem, gsem, ssem):
    cid = lax.axis_index("c")
    sid = lax.axis_index("s")
    cid_off = lax.broadcast_in_dim(cid * N, (L,), ()).astype(jnp.int32)

    pltpu.sync_copy(ai_hbm, ai_v)
    pltpu.sync_copy(aj_hbm, aj_v)

    def _zrow(r, carry):
        for j in range(DPH // L):
            rows_v[0, r, pl.ds(j * L, L)] = jnp.zeros((L,), jnp.float32)
        return carry
    lax.fori_loop(0, K, _zrow, 0)

    zbase = sid * ROWS_PT
    def _zcp(cn, carry):
        pltpu.sync_copy(rows_v.at[0],
                        acc_sh.at[pl.ds(zbase + cn * K, K)])
        return carry
    lax.fori_loop(0, ROWS_PT // K, _zcp, 0)
    plsc.subcore_barrier()

    ebase = sid * EPT

    def idx_copies(c, b):
        cb = ebase + c * K
        return (pltpu.make_async_copy(ei_hbm.at[:, pl.ds(cb, K)], ei_v.at[b],
                                      isem.at[b]),
                pltpu.make_async_copy(cos_hbm.at[pl.ds(cb, K)], cos_v.at[b],
                                      isem.at[b]))

    def start_idx(c, b):
        for d in idx_copies(c, b):
            d.start()

    def wait_idx(c, b):
        for d in idx_copies(c, b):
            d.wait()

    def gather_copy(b):
        return pltpu.make_async_copy(haug_hbm.at[ei_v.at[b, 0]],
                                     rows_v.at[b], gsem.at[b])

    def scatter_copy(b):
        return pltpu.make_async_copy(rows_v.at[b],
                                     acc_sh.at[ei_v.at[b, 1]], ssem.at[b])

    def compute_p(b):
        for gi in range(K // L):
            sl = pl.ds(gi * L, L)
            s_idx = ei_v[b, 0, sl]
            d_idx = ei_v[b, 1, sl]
            al = (plsc.load_gather(ai_v, [d_idx])
                  + plsc.load_gather(aj_v, [s_idx])) * cos_v[b, sl]
            al = jnp.where(al >= 0.0, al, al * jnp.float32(0.2))
            p_v[b, sl] = jnp.exp(al)

    def shift_src(b):
        for gi in range(K // L):
            sl = pl.ds(gi * L, L)
            ei_v[b, 0, sl] = ei_v[b, 0, sl] + cid_off

    def scale_rows(b):
        def _scale(g, carry2):
            pvec = p_v[b, pl.ds(g * L, L)]
            for i in range(L):
                pb = lax.broadcast_in_dim(pvec[i], (L,), ())
                r = g * L + i
                for j in range(DPH // L):
                    sj = pl.ds(j * L, L)
                    rows_v[b, r, sj] = rows_v[b, r, sj] * pb
            return carry2
        lax.fori_loop(0, K // L, _scale, 0)

    start_idx(0, 0)
    start_idx(1, 1)
    wait_idx(0, 0)
    compute_p(0)
    shift_src(0)
    gather_copy(0).start()

    def _round(t, carry):
        for r in range(NB):
            c = t * NB + r
            r1 = (r + 1) % NB
            r2 = (r + 2) % NB

            @pl.when(c >= NB - 2)
            def _():
                scatter_copy(r2).wait()

            @pl.when(c + 2 < NCHUNK)
            def _():
                start_idx(c + 2, r2)

            @pl.when(c + 1 < NCHUNK)
            def _():
                wait_idx(c + 1, r1)
                compute_p(r1)
                shift_src(r1)
                gather_copy(r1).start()

            gather_copy(r).wait()
            scale_rows(r)
            pltpu.async_copy(rows_v.at[r], acc_sh.at[ei_v.at[r, 1]],
                             ssem.at[r], add=True)
        return carry
    lax.fori_loop(0, NCHUNK // NB, _round, 0)

    for b in ((NCHUNK - 3) % NB, (NCHUNK - 2) % NB, (NCHUNK - 1) % NB):
        scatter_copy(b).wait()

    plsc.subcore_barrier()
    pltpu.sync_copy(acc_sh.at[pl.ds(sid * ROWS_PT, ROWS_PT)],
                    acc_hbm.at[cid, pl.ds(sid * ROWS_PT, ROWS_PT)])


@functools.cache
def _make_sc_edge():
    return functools.partial(
        pl.kernel,
        out_type=jax.ShapeDtypeStruct((NC, NP, DPH), jnp.float32),
        mesh=plsc.VectorSubcoreMesh(core_axis_name="c", subcore_axis_name="s",
                                    num_cores=NC, num_subcores=NS),
        compiler_params=pltpu.CompilerParams(needs_layout_passes=False,
                                             use_tc_tiling_on_sc=False),
        scratch_types=[
            pltpu.VMEM((N,), jnp.float32),
            pltpu.VMEM((N,), jnp.float32),
            pltpu.VMEM((NB, 2, K), jnp.int32),
            pltpu.VMEM((NB, K), jnp.float32),
            pltpu.VMEM((NB, K), jnp.float32),
            pltpu.VMEM((NB, K, DPH), jnp.float32),
            pltpu.VMEM_SHARED((NP, DPH), jnp.float32),
            pltpu.SemaphoreType.DMA((NB,)),
            pltpu.SemaphoreType.DMA((NB,)),
            pltpu.SemaphoreType.DMA((NB,)),
        ],
    )(_sc_edge_body)


def _tc2_body(acc_ref, bias_ref, gamma_ref, beta_ref, y_ref,
              out0_ref, sum_ref, sq_ref):
    p = pl.program_id(0)
    i = pl.program_id(1)

    @pl.when(p == 0)
    def _():
        a0 = acc_ref[0]
        a1 = acc_ref[1]
        num = jnp.concatenate([a0[:, :DH], a1[:, :DH]], axis=1)
        den = a0[:, DEN:DEN + 1]
        o = num / (den + 1e-16) + bias_ref[...]
        out0_ref[pl.ds(i * y_ref.shape[0], y_ref.shape[0]), :] = o

        @pl.when(i == 0)
        def _():
            sum_ref[...] = jnp.zeros_like(sum_ref)
            sq_ref[...] = jnp.zeros_like(sq_ref)

        sum_ref[...] += jnp.sum(o, axis=0, keepdims=True)
        sq_ref[...] += jnp.sum(o * o, axis=0, keepdims=True)

    @pl.when(p == 1)
    def _():
        mean = sum_ref[...] / jnp.float32(N)
        var = sq_ref[...] / jnp.float32(N) - mean * mean
        inv = lax.rsqrt(var + 1e-5)
        o = out0_ref[pl.ds(i * y_ref.shape[0], y_ref.shape[0]), :]
        y = (o - mean) * inv * gamma_ref[...] + beta_ref[...]
        y_ref[...] = jnp.maximum(y, 0.0)


def _tc2(acc, bias_row, gamma_row, beta_row):
    bm = 1000
    return pl.pallas_call(
        _tc2_body,
        grid=(2, N // bm),
        in_specs=[
            pl.BlockSpec((NC, bm, DPH), lambda p, i: (0, i * (1 - p), 0)),
            pl.BlockSpec((1, D), lambda p, i: (0, 0)),
            pl.BlockSpec((1, D), lambda p, i: (0, 0)),
            pl.BlockSpec((1, D), lambda p, i: (0, 0)),
        ],
        out_specs=pl.BlockSpec((bm, D), lambda p, i: (i * p, 0)),
        out_shape=jax.ShapeDtypeStruct((N, D), jnp.float32),
        scratch_shapes=[
            pltpu.VMEM((N, D), jnp.float32),
            pltpu.VMEM((1, D), jnp.float32),
            pltpu.VMEM((1, D), jnp.float32),
        ],
    )(acc, bias_row, gamma_row, beta_row)


def kernel(x, edge_index, cos_topk, embedding, W, att_i, att_j,
           att_em_i, att_em_j, bias, gamma, beta):
    att_ij = jnp.stack([att_i, att_j], axis=1)
    att_em_ij = jnp.stack([att_em_i, att_em_j], axis=1)
    haug, a2 = _tc1(x, embedding, W, att_ij, att_em_ij)
    ai = a2[:, 0]
    aj = a2[:, 1]
    acc = _make_sc_edge()(edge_index, cos_topk, ai, aj,
                          haug.reshape(2 * N, DPH))
    return _tc2(acc, bias.reshape(1, D), gamma.reshape(1, D),
                beta.reshape(1, D))
